# baseline (speedup 1.0000x reference)
import jax
import jax.numpy as jnp
from jax.experimental import pallas as pl
from jax.experimental.pallas import tpu as pltpu

SCORE_T = 0.5
NMS_T = 0.5


def _nms_body(b_ref, bT_ref, sc_ref, sr_ref, lab_ref,
              bx_ref, lb_ref, so_ref, kp_ref, pm_ref, src_ref,
              iou_ref):
    n = b_ref.shape[0]
    i0 = jax.lax.broadcasted_iota(jnp.int32, (n, n), 0)
    i1 = jax.lax.broadcasted_iota(jnp.int32, (n, n), 1)
    sa = sc_ref[...]
    sb = sr_ref[...]

    cmp_ij = (sb > sa) | ((sb == sa) & (i1 < i0))
    rank_c = jnp.sum(cmp_ij.astype(jnp.int32), axis=1, keepdims=True)
    cmp_ji = (sa > sb) | ((sa == sb) & (i0 < i1))
    rank_r = jnp.sum(cmp_ji.astype(jnp.int32), axis=0, keepdims=True)

    eq = rank_r == i0
    eqT = rank_c == i1
    eqf = eq.astype(jnp.float32)
    eqTf = eqT.astype(jnp.float32)

    def gcol(row_vals):
        return jnp.sum(eqf * row_vals, axis=1, keepdims=True)

    def grow(col_vals):
        return jnp.sum(eqTf * col_vals, axis=0, keepdims=True)

    x1_c = gcol(bT_ref[0:1, :]); y1_c = gcol(bT_ref[1:2, :])
    x2_c = gcol(bT_ref[2:3, :]); y2_c = gcol(bT_ref[3:4, :])
    x1_r = grow(b_ref[:, 0:1]); y1_r = grow(b_ref[:, 1:2])
    x2_r = grow(b_ref[:, 2:3]); y2_r = grow(b_ref[:, 3:4])
    s_c = gcol(sb)
    s_r = grow(sa)
    lab_c = jnp.sum(eq.astype(jnp.int32) * lab_ref[...], axis=1, keepdims=True)
    perm_c = jnp.sum(eq.astype(jnp.int32) * i1, axis=1, keepdims=True)
    perm_r = jnp.sum(eqT.astype(jnp.int32) * i0, axis=0, keepdims=True)

    xx1 = jnp.maximum(x1_c, x1_r)
    yy1 = jnp.maximum(y1_c, y1_r)
    xx2 = jnp.minimum(x2_c, x2_r)
    yy2 = jnp.minimum(y2_c, y2_r)
    inter = jnp.maximum(xx2 - xx1, 0.0) * jnp.maximum(yy2 - yy1, 0.0)
    area_c = (x2_c - x1_c) * (y2_c - y1_c)
    area_r = (x2_r - x1_r) * (y2_r - y1_r)
    iou_ref[...] = inter / (area_c + area_r - inter)

    valid = s_r > SCORE_T
    kcount = jnp.sum(valid.astype(jnp.int32))
    nchunks = (kcount + 7) // 8
    idxr = jax.lax.broadcasted_iota(jnp.int32, (1, n), 1)

    def chunk_body(c, keep):
        base = pl.multiple_of(c * 8, 8)
        chunk = iou_ref[pl.ds(base, 8), :]
        for t in range(8):
            i = c * 8 + t
            row = chunk[t:t + 1, :]
            sup = jnp.any((idxr < i) & (keep != 0) & (row > NMS_T))
            keep = jnp.where((idxr == i) & sup, 0, keep)
        return keep

    keep_i = jax.lax.fori_loop(0, nchunks, chunk_body,
                               valid.astype(jnp.int32))
    keep = keep_i != 0

    keep_ci = jnp.sum(((i0 == i1) & keep).astype(jnp.int32),
                      axis=1, keepdims=True)
    keep_cf = keep_ci.astype(jnp.float32)

    bx_ref[:, 0:1] = x1_c * keep_cf
    bx_ref[:, 1:2] = y1_c * keep_cf
    bx_ref[:, 2:3] = x2_c * keep_cf
    bx_ref[:, 3:4] = y2_c * keep_cf
    lb_ref[...] = lab_c * keep_ci
    so_ref[...] = s_c * keep_cf
    kp_ref[...] = keep_ci
    pm_ref[...] = perm_c

    t_col = jnp.max(jnp.where((i1 <= i0) & keep, i1, -1),
                    axis=1, keepdims=True)
    sel = (i1 == t_col).astype(jnp.int32)
    src_col = jnp.sum(sel * perm_r, axis=1, keepdims=True)
    src_ref[...] = jnp.where(t_col < 0, perm_c, src_col)


_DMA_WINDOW = 16


def _mask_body(src_ref, keep_ref, m_ref, o_ref, z_ref, sem):
    n = o_ref.shape[0]
    z_ref[...] = jnp.zeros_like(z_ref)

    def wait_one(r):
        pltpu.make_async_copy(z_ref, o_ref.at[r], sem).wait()

    def body(r, carry):
        @pl.when(keep_ref[r] != 0)
        def _():
            pltpu.make_async_copy(m_ref.at[src_ref[r]], o_ref.at[r],
                                  sem).start()

        @pl.when(keep_ref[r] == 0)
        def _():
            pltpu.make_async_copy(z_ref, o_ref.at[r], sem).start()

        @pl.when(r >= _DMA_WINDOW)
        def _():
            wait_one(r - _DMA_WINDOW)
        return carry

    jax.lax.fori_loop(0, n, body, 0)

    def tail(r, carry):
        wait_one(r)
        return carry

    jax.lax.fori_loop(n - _DMA_WINDOW, n, tail, 0)


def kernel(boxes, scores, labels, masks):
    n = boxes.shape[0]
    h, w = masks.shape[2], masks.shape[3]

    bx, lb, so, kp, pm, src = pl.pallas_call(
        _nms_body,
        out_shape=[
            jax.ShapeDtypeStruct((n, 4), jnp.float32),
            jax.ShapeDtypeStruct((n, 1), jnp.int32),
            jax.ShapeDtypeStruct((n, 1), jnp.float32),
            jax.ShapeDtypeStruct((n, 1), jnp.int32),
            jax.ShapeDtypeStruct((n, 1), jnp.int32),
            jax.ShapeDtypeStruct((n, 1), jnp.int32),
        ],
        scratch_shapes=[pltpu.VMEM((n, n), jnp.float32)],
        name="nms_sort",
    )(boxes, boxes.T, scores[:, None], scores[None, :], labels[None, :])

    keep_i = kp[:, 0]
    masks_out = pl.pallas_call(
        _mask_body,
        grid_spec=pltpu.PrefetchScalarGridSpec(
            num_scalar_prefetch=2,
            grid=(),
            in_specs=[pl.BlockSpec(memory_space=pl.ANY)],
            out_specs=pl.BlockSpec(memory_space=pl.ANY),
            scratch_shapes=[pltpu.VMEM((h, w), jnp.float32),
                            pltpu.SemaphoreType.DMA],
        ),
        out_shape=jax.ShapeDtypeStruct((n, h, w), jnp.float32),
        name="mask_gather",
    )(src[:, 0], keep_i, masks.reshape(n, h, w))

    return (bx, lb[:, 0], so[:, 0], masks_out.reshape(masks.shape),
            keep_i.astype(jnp.bool_))


# speedup vs baseline: 1.0001x; 1.0001x over previous
import jax
import jax.numpy as jnp
from jax.experimental import pallas as pl
from jax.experimental.pallas import tpu as pltpu

SCORE_T = 0.5
NMS_T = 0.5


def _nms_body(b_ref, bT_ref, sc_ref, sr_ref, lab_ref,
              bx_ref, lb_ref, so_ref, kp_ref, pm_ref, src_ref,
              iou_ref):
    n = b_ref.shape[0]
    i0 = jax.lax.broadcasted_iota(jnp.int32, (n, n), 0)
    i1 = jax.lax.broadcasted_iota(jnp.int32, (n, n), 1)
    sa = sc_ref[...]
    sb = sr_ref[...]

    cmp_ij = (sb > sa) | ((sb == sa) & (i1 < i0))
    rank_c = jnp.sum(cmp_ij.astype(jnp.int32), axis=1, keepdims=True)
    cmp_ji = (sa > sb) | ((sa == sb) & (i0 < i1))
    rank_r = jnp.sum(cmp_ji.astype(jnp.int32), axis=0, keepdims=True)

    eq = rank_r == i0
    eqT = rank_c == i1
    eqf = eq.astype(jnp.float32)
    eqTf = eqT.astype(jnp.float32)

    def gcol(row_vals):
        return jnp.sum(eqf * row_vals, axis=1, keepdims=True)

    def grow(col_vals):
        return jnp.sum(eqTf * col_vals, axis=0, keepdims=True)

    x1_c = gcol(bT_ref[0:1, :]); y1_c = gcol(bT_ref[1:2, :])
    x2_c = gcol(bT_ref[2:3, :]); y2_c = gcol(bT_ref[3:4, :])
    x1_r = grow(b_ref[:, 0:1]); y1_r = grow(b_ref[:, 1:2])
    x2_r = grow(b_ref[:, 2:3]); y2_r = grow(b_ref[:, 3:4])
    s_c = gcol(sb)
    s_r = grow(sa)
    lab_c = jnp.sum(eq.astype(jnp.int32) * lab_ref[...], axis=1, keepdims=True)
    perm_c = jnp.sum(eq.astype(jnp.int32) * i1, axis=1, keepdims=True)
    perm_r = jnp.sum(eqT.astype(jnp.int32) * i0, axis=0, keepdims=True)

    xx1 = jnp.maximum(x1_c, x1_r)
    yy1 = jnp.maximum(y1_c, y1_r)
    xx2 = jnp.minimum(x2_c, x2_r)
    yy2 = jnp.minimum(y2_c, y2_r)
    inter = jnp.maximum(xx2 - xx1, 0.0) * jnp.maximum(yy2 - yy1, 0.0)
    area_c = (x2_c - x1_c) * (y2_c - y1_c)
    area_r = (x2_r - x1_r) * (y2_r - y1_r)
    iou_ref[...] = inter / (area_c + area_r - inter)

    valid = s_r > SCORE_T
    kcount = jnp.sum(valid.astype(jnp.int32))
    nchunks = (kcount + 7) // 8
    idxr = jax.lax.broadcasted_iota(jnp.int32, (1, n), 1)

    def chunk_body(c, keep):
        base = pl.multiple_of(c * 8, 8)
        chunk = iou_ref[pl.ds(base, 8), :]
        for t in range(8):
            i = c * 8 + t
            row = chunk[t:t + 1, :]
            sup = jnp.any((idxr < i) & (keep != 0) & (row > NMS_T))
            keep = jnp.where((idxr == i) & sup, 0, keep)
        return keep

    keep_i = jax.lax.fori_loop(0, nchunks, chunk_body,
                               valid.astype(jnp.int32))
    keep = keep_i != 0

    keep_ci = jnp.sum(((i0 == i1) & keep).astype(jnp.int32),
                      axis=1, keepdims=True)
    keep_cf = keep_ci.astype(jnp.float32)

    bx_ref[:, 0:1] = x1_c * keep_cf
    bx_ref[:, 1:2] = y1_c * keep_cf
    bx_ref[:, 2:3] = x2_c * keep_cf
    bx_ref[:, 3:4] = y2_c * keep_cf
    lb_ref[...] = lab_c * keep_ci
    so_ref[...] = s_c * keep_cf
    kp_ref[...] = keep_ci
    pm_ref[...] = perm_c

    t_col = jnp.max(jnp.where((i1 <= i0) & keep, i1, -1),
                    axis=1, keepdims=True)
    sel = (i1 == t_col).astype(jnp.int32)
    src_col = jnp.sum(sel * perm_r, axis=1, keepdims=True)
    src_ref[...] = jnp.where(t_col < 0, perm_c, src_col)


_DMA_WINDOW = 16


def _mask_body(src_ref, keep_ref, m_ref, o_ref, z_ref, sems):
    n = o_ref.shape[0]
    z_ref[...] = jnp.zeros_like(z_ref)

    def wait_one(r):
        slot = jax.lax.rem(r, _DMA_WINDOW)
        pltpu.make_async_copy(z_ref, o_ref.at[r], sems.at[slot]).wait()

    def body(r, carry):
        slot = jax.lax.rem(r, _DMA_WINDOW)

        @pl.when(keep_ref[r] != 0)
        def _():
            pltpu.make_async_copy(m_ref.at[src_ref[r]], o_ref.at[r],
                                  sems.at[slot]).start()

        @pl.when(keep_ref[r] == 0)
        def _():
            pltpu.make_async_copy(z_ref, o_ref.at[r], sems.at[slot]).start()

        @pl.when(r >= _DMA_WINDOW)
        def _():
            wait_one(r - _DMA_WINDOW)
        return carry

    jax.lax.fori_loop(0, n, body, 0)

    def tail(r, carry):
        wait_one(r)
        return carry

    jax.lax.fori_loop(n - _DMA_WINDOW, n, tail, 0)


def kernel(boxes, scores, labels, masks):
    n = boxes.shape[0]
    h, w = masks.shape[2], masks.shape[3]

    bx, lb, so, kp, pm, src = pl.pallas_call(
        _nms_body,
        out_shape=[
            jax.ShapeDtypeStruct((n, 4), jnp.float32),
            jax.ShapeDtypeStruct((n, 1), jnp.int32),
            jax.ShapeDtypeStruct((n, 1), jnp.float32),
            jax.ShapeDtypeStruct((n, 1), jnp.int32),
            jax.ShapeDtypeStruct((n, 1), jnp.int32),
            jax.ShapeDtypeStruct((n, 1), jnp.int32),
        ],
        scratch_shapes=[pltpu.VMEM((n, n), jnp.float32)],
        name="nms_sort",
    )(boxes, boxes.T, scores[:, None], scores[None, :], labels[None, :])

    keep_i = kp[:, 0]
    masks_out = pl.pallas_call(
        _mask_body,
        grid_spec=pltpu.PrefetchScalarGridSpec(
            num_scalar_prefetch=2,
            grid=(),
            in_specs=[pl.BlockSpec(memory_space=pl.ANY)],
            out_specs=pl.BlockSpec(memory_space=pl.ANY),
            scratch_shapes=[pltpu.VMEM((h, w), jnp.float32),
                            pltpu.SemaphoreType.DMA((_DMA_WINDOW,))],
        ),
        out_shape=jax.ShapeDtypeStruct((n, h, w), jnp.float32),
        name="mask_gather",
    )(src[:, 0], keep_i, masks.reshape(n, h, w))

    return (bx, lb[:, 0], so[:, 0], masks_out.reshape(masks.shape),
            keep_i.astype(jnp.bool_))


# speedup vs baseline: 22.3475x; 22.3462x over previous
import jax
import jax.numpy as jnp
from jax.experimental import pallas as pl
from jax.experimental.pallas import tpu as pltpu

SCORE_T = 0.5
NMS_T = 0.5


def _nms_body(b_ref, bT_ref, sc_ref, sr_ref, lab_ref,
              bx_ref, lb_ref, so_ref, kp_ref, pm_ref, src_ref,
              iou_ref):
    n = b_ref.shape[0]
    i0 = jax.lax.broadcasted_iota(jnp.int32, (n, n), 0)
    i1 = jax.lax.broadcasted_iota(jnp.int32, (n, n), 1)
    sa = sc_ref[...]
    sb = sr_ref[...]

    cmp_ij = (sb > sa) | ((sb == sa) & (i1 < i0))
    rank_c = jnp.sum(cmp_ij.astype(jnp.int32), axis=1, keepdims=True)
    cmp_ji = (sa > sb) | ((sa == sb) & (i0 < i1))
    rank_r = jnp.sum(cmp_ji.astype(jnp.int32), axis=0, keepdims=True)

    eq = rank_r == i0
    eqT = rank_c == i1
    eqf = eq.astype(jnp.float32)
    eqTf = eqT.astype(jnp.float32)

    def gcol(row_vals):
        return jnp.sum(eqf * row_vals, axis=1, keepdims=True)

    def grow(col_vals):
        return jnp.sum(eqTf * col_vals, axis=0, keepdims=True)

    x1_c = gcol(bT_ref[0:1, :]); y1_c = gcol(bT_ref[1:2, :])
    x2_c = gcol(bT_ref[2:3, :]); y2_c = gcol(bT_ref[3:4, :])
    x1_r = grow(b_ref[:, 0:1]); y1_r = grow(b_ref[:, 1:2])
    x2_r = grow(b_ref[:, 2:3]); y2_r = grow(b_ref[:, 3:4])
    s_c = gcol(sb)
    s_r = grow(sa)
    lab_c = jnp.sum(eq.astype(jnp.int32) * lab_ref[...], axis=1, keepdims=True)
    perm_c = jnp.sum(eq.astype(jnp.int32) * i1, axis=1, keepdims=True)
    perm_r = jnp.sum(eqT.astype(jnp.int32) * i0, axis=0, keepdims=True)

    xx1 = jnp.maximum(x1_c, x1_r)
    yy1 = jnp.maximum(y1_c, y1_r)
    xx2 = jnp.minimum(x2_c, x2_r)
    yy2 = jnp.minimum(y2_c, y2_r)
    inter = jnp.maximum(xx2 - xx1, 0.0) * jnp.maximum(yy2 - yy1, 0.0)
    area_c = (x2_c - x1_c) * (y2_c - y1_c)
    area_r = (x2_r - x1_r) * (y2_r - y1_r)
    iou_ref[...] = inter / (area_c + area_r - inter)

    valid = s_r > SCORE_T
    kcount = jnp.sum(valid.astype(jnp.int32))
    nchunks = (kcount + 7) // 8
    idxr = jax.lax.broadcasted_iota(jnp.int32, (1, n), 1)

    def chunk_body(c, keep):
        base = pl.multiple_of(c * 8, 8)
        chunk = iou_ref[pl.ds(base, 8), :]
        for t in range(8):
            i = c * 8 + t
            row = chunk[t:t + 1, :]
            sup = jnp.any((idxr < i) & (keep != 0) & (row > NMS_T))
            keep = jnp.where((idxr == i) & sup, 0, keep)
        return keep

    keep_i = jax.lax.fori_loop(0, nchunks, chunk_body,
                               valid.astype(jnp.int32))
    keep = keep_i != 0

    keep_ci = jnp.sum(((i0 == i1) & keep).astype(jnp.int32),
                      axis=1, keepdims=True)
    keep_cf = keep_ci.astype(jnp.float32)

    bx_ref[:, 0:1] = x1_c * keep_cf
    bx_ref[:, 1:2] = y1_c * keep_cf
    bx_ref[:, 2:3] = x2_c * keep_cf
    bx_ref[:, 3:4] = y2_c * keep_cf
    lb_ref[...] = lab_c * keep_ci
    so_ref[...] = s_c * keep_cf
    kp_ref[...] = keep_ci
    pm_ref[...] = perm_c

    t_col = jnp.max(jnp.where((i1 <= i0) & keep, i1, -1),
                    axis=1, keepdims=True)
    sel = (i1 == t_col).astype(jnp.int32)
    src_col = jnp.sum(sel * perm_r, axis=1, keepdims=True)
    src_ref[...] = jnp.where(t_col < 0, perm_c, src_col)


_ROWS_PER_STEP = 8


def _mask_body(src_ref, keep_ref, *refs):
    g = _ROWS_PER_STEP
    o_ref = refs[g]
    r = pl.program_id(0)
    for j in range(g):
        k = keep_ref[r * g + j].astype(jnp.float32)
        o_ref[j] = refs[j][0] * k


def kernel(boxes, scores, labels, masks):
    n = boxes.shape[0]
    h, w = masks.shape[2], masks.shape[3]

    bx, lb, so, kp, pm, src = pl.pallas_call(
        _nms_body,
        out_shape=[
            jax.ShapeDtypeStruct((n, 4), jnp.float32),
            jax.ShapeDtypeStruct((n, 1), jnp.int32),
            jax.ShapeDtypeStruct((n, 1), jnp.float32),
            jax.ShapeDtypeStruct((n, 1), jnp.int32),
            jax.ShapeDtypeStruct((n, 1), jnp.int32),
            jax.ShapeDtypeStruct((n, 1), jnp.int32),
        ],
        scratch_shapes=[pltpu.VMEM((n, n), jnp.float32)],
        name="nms_sort",
    )(boxes, boxes.T, scores[:, None], scores[None, :], labels[None, :])

    keep_i = kp[:, 0]
    g = _ROWS_PER_STEP
    m3 = masks.reshape(n, h, w)

    def make_in_spec(j):
        return pl.BlockSpec((1, h, w),
                            lambda r, src, keep, j=j: (src[r * g + j], 0, 0))

    masks_out = pl.pallas_call(
        _mask_body,
        grid_spec=pltpu.PrefetchScalarGridSpec(
            num_scalar_prefetch=2,
            grid=(n // g,),
            in_specs=[make_in_spec(j) for j in range(g)],
            out_specs=pl.BlockSpec((g, h, w),
                                   lambda r, src, keep: (r, 0, 0)),
        ),
        out_shape=jax.ShapeDtypeStruct((n, h, w), jnp.float32),
        compiler_params=pltpu.CompilerParams(
            dimension_semantics=("arbitrary",)),
        name="mask_gather",
    )(src[:, 0], keep_i, *([m3] * g))

    return (bx, lb[:, 0], so[:, 0], masks_out.reshape(masks.shape),
            keep_i.astype(jnp.bool_))


# speedup vs baseline: 23.6540x; 1.0585x over previous
import jax
import jax.numpy as jnp
from jax.experimental import pallas as pl
from jax.experimental.pallas import tpu as pltpu

SCORE_T = 0.5
NMS_T = 0.5


def _nms_body(b_ref, bT_ref, sc_ref, sr_ref, lab_ref,
              bx_ref, lb_ref, so_ref, kp_ref, pm_ref, src_ref,
              iou_ref):
    n = b_ref.shape[0]
    i0 = jax.lax.broadcasted_iota(jnp.int32, (n, n), 0)
    i1 = jax.lax.broadcasted_iota(jnp.int32, (n, n), 1)
    sa = sc_ref[...]
    sb = sr_ref[...]

    cmp_ij = (sb > sa) | ((sb == sa) & (i1 < i0))
    rank_c = jnp.sum(cmp_ij.astype(jnp.int32), axis=1, keepdims=True)
    cmp_ji = (sa > sb) | ((sa == sb) & (i0 < i1))
    rank_r = jnp.sum(cmp_ji.astype(jnp.int32), axis=0, keepdims=True)

    eq = rank_r == i0
    eqT = rank_c == i1
    eqf = eq.astype(jnp.float32)
    eqTf = eqT.astype(jnp.float32)

    def gcol(row_vals):
        return jnp.sum(eqf * row_vals, axis=1, keepdims=True)

    def grow(col_vals):
        return jnp.sum(eqTf * col_vals, axis=0, keepdims=True)

    x1_c = gcol(bT_ref[0:1, :]); y1_c = gcol(bT_ref[1:2, :])
    x2_c = gcol(bT_ref[2:3, :]); y2_c = gcol(bT_ref[3:4, :])
    x1_r = grow(b_ref[:, 0:1]); y1_r = grow(b_ref[:, 1:2])
    x2_r = grow(b_ref[:, 2:3]); y2_r = grow(b_ref[:, 3:4])
    s_c = gcol(sb)
    s_r = grow(sa)
    lab_c = jnp.sum(eq.astype(jnp.int32) * lab_ref[...], axis=1, keepdims=True)
    perm_c = jnp.sum(eq.astype(jnp.int32) * i1, axis=1, keepdims=True)
    perm_r = jnp.sum(eqT.astype(jnp.int32) * i0, axis=0, keepdims=True)

    xx1 = jnp.maximum(x1_c, x1_r)
    yy1 = jnp.maximum(y1_c, y1_r)
    xx2 = jnp.minimum(x2_c, x2_r)
    yy2 = jnp.minimum(y2_c, y2_r)
    inter = jnp.maximum(xx2 - xx1, 0.0) * jnp.maximum(yy2 - yy1, 0.0)
    area_c = (x2_c - x1_c) * (y2_c - y1_c)
    area_r = (x2_r - x1_r) * (y2_r - y1_r)
    iou_ref[...] = inter / (area_c + area_r - inter)

    valid = s_r > SCORE_T
    kcount = jnp.sum(valid.astype(jnp.int32))
    nchunks = (kcount + 7) // 8
    idxr = jax.lax.broadcasted_iota(jnp.int32, (1, n), 1)

    def chunk_body(c, keep):
        base = pl.multiple_of(c * 8, 8)
        chunk = iou_ref[pl.ds(base, 8), :]
        for t in range(8):
            i = c * 8 + t
            row = chunk[t:t + 1, :]
            sup = jnp.any((idxr < i) & (keep != 0) & (row > NMS_T))
            keep = jnp.where((idxr == i) & sup, 0, keep)
        return keep

    keep_i = jax.lax.fori_loop(0, nchunks, chunk_body,
                               valid.astype(jnp.int32))
    keep = keep_i != 0

    keep_ci = jnp.sum(((i0 == i1) & keep).astype(jnp.int32),
                      axis=1, keepdims=True)
    keep_cf = keep_ci.astype(jnp.float32)

    bx_ref[:, 0:1] = x1_c * keep_cf
    bx_ref[:, 1:2] = y1_c * keep_cf
    bx_ref[:, 2:3] = x2_c * keep_cf
    bx_ref[:, 3:4] = y2_c * keep_cf
    lb_ref[...] = lab_c * keep_ci
    so_ref[...] = s_c * keep_cf
    kp_ref[...] = keep_ci
    pm_ref[...] = perm_c

    g = _ROWS_PER_STEP
    lane_ok = (i1 <= i0) & (((i0 - i1) & (g - 1)) == 0) & keep
    t_col = jnp.max(jnp.where(lane_ok, i1, -1),
                    axis=1, keepdims=True)
    sel = (i1 == t_col).astype(jnp.int32)
    src_col = jnp.sum(sel * perm_r, axis=1, keepdims=True)
    src_ref[...] = jnp.where(t_col < 0, perm_c, src_col)


_ROWS_PER_STEP = 8


def _mask_body(src_ref, keep_ref, *refs):
    g = _ROWS_PER_STEP
    o_ref = refs[g]
    r = pl.program_id(0)
    for j in range(g):
        k = keep_ref[r * g + j].astype(jnp.float32)
        o_ref[j] = refs[j][0] * k


def kernel(boxes, scores, labels, masks):
    n = boxes.shape[0]
    h, w = masks.shape[2], masks.shape[3]

    bx, lb, so, kp, pm, src = pl.pallas_call(
        _nms_body,
        out_shape=[
            jax.ShapeDtypeStruct((n, 4), jnp.float32),
            jax.ShapeDtypeStruct((n, 1), jnp.int32),
            jax.ShapeDtypeStruct((n, 1), jnp.float32),
            jax.ShapeDtypeStruct((n, 1), jnp.int32),
            jax.ShapeDtypeStruct((n, 1), jnp.int32),
            jax.ShapeDtypeStruct((n, 1), jnp.int32),
        ],
        scratch_shapes=[pltpu.VMEM((n, n), jnp.float32)],
        name="nms_sort",
    )(boxes, boxes.T, scores[:, None], scores[None, :], labels[None, :])

    keep_i = kp[:, 0]
    g = _ROWS_PER_STEP
    m3 = masks.reshape(n, h, w)

    def make_in_spec(j):
        return pl.BlockSpec((1, h, w),
                            lambda r, src, keep, j=j: (src[r * g + j], 0, 0))

    masks_out = pl.pallas_call(
        _mask_body,
        grid_spec=pltpu.PrefetchScalarGridSpec(
            num_scalar_prefetch=2,
            grid=(n // g,),
            in_specs=[make_in_spec(j) for j in range(g)],
            out_specs=pl.BlockSpec((g, h, w),
                                   lambda r, src, keep: (r, 0, 0)),
        ),
        out_shape=jax.ShapeDtypeStruct((n, h, w), jnp.float32),
        compiler_params=pltpu.CompilerParams(
            dimension_semantics=("arbitrary",)),
        name="mask_gather",
    )(src[:, 0], keep_i, *([m3] * g))

    return (bx, lb[:, 0], so[:, 0], masks_out.reshape(masks.shape),
            keep_i.astype(jnp.bool_))
